# baseline (speedup 1.0000x reference)
import functools

import jax
import jax.numpy as jnp
from jax import lax
from jax.experimental import pallas as pl
from jax.experimental.pallas import tpu as pltpu
from jax.experimental.pallas import tpu_sc as plsc

B, P, C = 16, 512, 512
L = 2048
LANES = 16

NUM_CORES = 2
NUM_SUBCORES = 16
NW = NUM_CORES * NUM_SUBCORES
FRAMES_PER_TILE = (B * L) // NW
HALF = FRAMES_PER_TILE
CH = 64
NCHUNK = FRAMES_PER_TILE // CH


def _sc_expand(table, durations):
    mesh = plsc.VectorSubcoreMesh(core_axis_name="c", subcore_axis_name="s")

    @functools.partial(
        pl.kernel,
        mesh=mesh,
        out_type=jax.ShapeDtypeStruct((B * L, C), jnp.float32),
        compiler_params=pltpu.CompilerParams(needs_layout_passes=False),
        scratch_types=[
            pltpu.VMEM((P,), jnp.int32),
            pltpu.VMEM((HALF,), jnp.int32),
            pltpu.VMEM((HALF,), jnp.int32),
            pltpu.VMEM((CH, C), jnp.float32),
            pltpu.VMEM((CH, C), jnp.float32),
            pltpu.VMEM((CH, C), jnp.float32),
            pltpu.SemaphoreType.DMA,
            pltpu.SemaphoreType.DMA,
            pltpu.SemaphoreType.DMA,
            pltpu.SemaphoreType.DMA,
            pltpu.SemaphoreType.DMA,
        ],
    )
    def k(table_hbm, dur_hbm, out_hbm, dur_v, a_v, idx_v, buf0, buf1,
          zbuf, gs0, gs1, ws0, ws1, wsp):
        sid = lax.axis_index("s")
        cid = lax.axis_index("c")
        tile = cid * 8 + (sid % 8) + (sid // 8) * 16
        b = tile % B
        lo = (tile // B) * HALF

        pltpu.async_copy(dur_hbm.at[pl.ds(b * P, P)], dur_v, gs0)

        def zbuf_body(r, _):
            for j in range(C // LANES):
                zbuf[r, pl.ds(j * LANES, LANES)] = jnp.zeros(
                    (LANES,), jnp.float32)
            return 0

        lax.fori_loop(0, CH, zbuf_body, 0)
        pltpu.make_async_copy(dur_hbm.at[pl.ds(b * P, P)], dur_v, gs0).wait()

        def sum_body(i, acc):
            return acc + jnp.sum(dur_v[pl.ds(i * LANES, LANES)])

        total = lax.fori_loop(0, P // LANES, sum_body, jnp.int32(0))
        total_rel = jnp.clip(total - lo, 0, HALF)
        n_real_ch = (total_rel + CH - 1) // CH
        n_pad = NCHUNK - n_real_ch

        out_base = b * L + lo

        def out_slice(c):
            return out_hbm.at[pl.ds(out_base + c * CH, CH)]

        def pad_issue(i, _):
            pltpu.async_copy(zbuf, out_slice(n_real_ch + i), wsp)
            return 0

        n_pad_pre = jnp.minimum(n_pad, 4)
        lax.fori_loop(0, n_pad_pre, pad_issue, 0)

        def zero_body(i, _):
            a_v[pl.ds(i * LANES, LANES)] = jnp.zeros((LANES,), jnp.int32)
            return 0

        lax.fori_loop(0, HALF // LANES, zero_body, 0)

        def p1_body(i, carry):
            csum_in, maxc = carry
            d = dur_v[pl.ds(i * LANES, LANES)]
            cs = plsc.cumsum(d) + csum_in
            start = cs - d
            pvec = lax.iota(jnp.int32, LANES) + i * LANES
            pos = jnp.clip(start - lo, 0, HALF - 1)
            m = (d > 0) & (start >= lo) & (start < lo + HALF)
            plsc.store_scatter(a_v, [pos], pvec, mask=m)
            before = jnp.where((d > 0) & (start < lo), pvec, 0)
            return csum_in + jnp.sum(d), jnp.maximum(maxc, jnp.max(before))

        _, maxc = lax.fori_loop(0, P // LANES, p1_body,
                                (jnp.int32(0), jnp.int32(0)))

        def p2_body(i, mc):
            a = a_v[pl.ds(i * LANES, LANES)]
            vals = jnp.maximum(plsc.cummax(a), mc)
            t = lo + i * LANES + lax.iota(jnp.int32, LANES)
            rows = jnp.where(t < total, vals, t & (P - 1)) + b * P
            idx_v[pl.ds(i * LANES, LANES)] = rows
            return jnp.max(vals)

        lax.fori_loop(0, n_real_ch * (CH // LANES), p2_body, maxc)

        bufs = (buf0, buf1)
        gsems = (gs0, gs1)
        wsems = (ws0, ws1)

        def idx_slice(c):
            return table_hbm.at[idx_v.at[pl.ds(c * CH, CH)]]

        def has_real(c):
            return c * CH < total_rel

        def drain_real(c):
            @pl.when(has_real(c))
            def _():
                pltpu.make_async_copy(zbuf, out_slice(c), wsems[c % 2]).wait()

        def start_chunk(c):
            @pl.when(has_real(c))
            def _():
                pltpu.async_copy(idx_slice(c), bufs[c % 2], gsems[c % 2])

        def finish_chunk(c):
            nreal = jnp.clip(total_rel - c * CH, 0, CH)

            @pl.when(has_real(c))
            def _():
                pltpu.make_async_copy(
                    idx_slice(c), bufs[c % 2], gsems[c % 2]).wait()

                def ztail(r, _):
                    for j in range(C // LANES):
                        bufs[c % 2][r, pl.ds(j * LANES, LANES)] = jnp.zeros(
                            (LANES,), jnp.float32)
                    return 0

                lax.fori_loop(nreal, CH, ztail, 0)
                pltpu.async_copy(bufs[c % 2], out_slice(c), wsems[c % 2])

        for ch in range(NCHUNK):
            if ch >= 2:
                drain_real(ch - 2)
            start_chunk(ch)
            if ch >= 1:
                finish_chunk(ch - 1)
        finish_chunk(NCHUNK - 1)

        lax.fori_loop(n_pad_pre, n_pad, pad_issue, 0)
        drain_real(NCHUNK - 2)
        drain_real(NCHUNK - 1)

        def pad_wait(i, _):
            pltpu.make_async_copy(zbuf, out_slice(0), wsp).wait()
            return 0

        lax.fori_loop(0, n_pad, pad_wait, 0)

    return k(table, durations)


def kernel(encoder_output, log_durations):
    ld = log_durations[..., 0]
    m = (ld > 0).astype(jnp.int32)
    durations = (jnp.floor(jnp.power(2.0, ld) + 0.0001)
                 .astype(jnp.int32) * m)
    table = encoder_output.reshape(B * P, C)
    out = _sc_expand(table, durations.reshape(-1))
    return out.reshape(B, L, C)


# speedup vs baseline: 1.0202x; 1.0202x over previous
import functools

import jax
import jax.numpy as jnp
from jax import lax
from jax.experimental import pallas as pl
from jax.experimental.pallas import tpu as pltpu
from jax.experimental.pallas import tpu_sc as plsc

B, P, C = 16, 512, 512
L = 2048
LANES = 16

NUM_CORES = 2
NUM_SUBCORES = 16
NW = NUM_CORES * NUM_SUBCORES
FRAMES_PER_TILE = (B * L) // NW
HALF = FRAMES_PER_TILE
CH = 64
NCHUNK = FRAMES_PER_TILE // CH


def _sc_expand(table, durations):
    mesh = plsc.VectorSubcoreMesh(core_axis_name="c", subcore_axis_name="s")

    @functools.partial(
        pl.kernel,
        mesh=mesh,
        out_type=jax.ShapeDtypeStruct((B * L, C), jnp.float32),
        compiler_params=pltpu.CompilerParams(needs_layout_passes=False),
        scratch_types=[
            pltpu.VMEM((P,), jnp.int32),
            pltpu.VMEM((HALF,), jnp.int32),
            pltpu.VMEM((HALF,), jnp.int32),
            pltpu.VMEM((CH, C), jnp.float32),
            pltpu.VMEM((CH, C), jnp.float32),
            pltpu.VMEM((CH, C), jnp.float32),
            pltpu.SemaphoreType.DMA,
            pltpu.SemaphoreType.DMA,
            pltpu.SemaphoreType.DMA,
            pltpu.SemaphoreType.DMA,
            pltpu.SemaphoreType.DMA,
        ],
    )
    def k(table_hbm, dur_hbm, out_hbm, dur_v, a_v, idx_v, buf0, buf1,
          zbuf, gs0, gs1, ws0, ws1, wsp):
        sid = lax.axis_index("s")
        cid = lax.axis_index("c")
        tile = cid * 8 + (sid % 8) + (sid // 8) * 16
        b = tile % B

        pltpu.async_copy(dur_hbm.at[pl.ds(b * P, P)], dur_v, gs0)

        def zbuf_body(r, _):
            for j in range(C // LANES):
                zbuf[r, pl.ds(j * LANES, LANES)] = jnp.zeros(
                    (LANES,), jnp.float32)
            return 0

        lax.fori_loop(0, CH, zbuf_body, 0)
        pltpu.make_async_copy(dur_hbm.at[pl.ds(b * P, P)], dur_v, gs0).wait()

        def sum_body(i, acc):
            return acc + jnp.sum(dur_v[pl.ds(i * LANES, LANES)])

        total = lax.fori_loop(0, P // LANES, sum_body, jnp.int32(0))

        tot_l = jnp.minimum(total, L)
        rb = (tot_l + CH - 1) // CH
        r_t0 = (rb + 1) // 2
        half1 = tile // B
        rc = jnp.where(half1 == 0, r_t0, rb - r_t0)
        s0 = jnp.where(half1 == 0, 0, r_t0)
        n_pad = NCHUNK - rc
        p0 = jnp.where(half1 == 0, rb, rb + NCHUNK - r_t0)
        lo = s0 * CH

        def out_slice(c):
            return out_hbm.at[pl.ds(b * L + c * CH, CH)]

        def pad_issue(i, _):
            pltpu.async_copy(zbuf, out_slice(p0 + i), wsp)
            return 0

        n_pad_pre = jnp.minimum(n_pad, 4)
        lax.fori_loop(0, n_pad_pre, pad_issue, 0)

        def zero_body(i, _):
            a_v[pl.ds(i * LANES, LANES)] = jnp.zeros((LANES,), jnp.int32)
            return 0

        lax.fori_loop(0, HALF // LANES, zero_body, 0)

        def p1_body(i, carry):
            csum_in, maxc = carry
            d = dur_v[pl.ds(i * LANES, LANES)]
            cs = plsc.cumsum(d) + csum_in
            start = cs - d
            pvec = lax.iota(jnp.int32, LANES) + i * LANES
            pos = jnp.clip(start - lo, 0, HALF - 1)
            m = (d > 0) & (start >= lo) & (start < lo + HALF)
            plsc.store_scatter(a_v, [pos], pvec, mask=m)
            before = jnp.where((d > 0) & (start < lo), pvec, 0)
            return csum_in + jnp.sum(d), jnp.maximum(maxc, jnp.max(before))

        _, maxc = lax.fori_loop(0, P // LANES, p1_body,
                                (jnp.int32(0), jnp.int32(0)))

        def p2_body(i, mc):
            a = a_v[pl.ds(i * LANES, LANES)]
            vals = jnp.maximum(plsc.cummax(a), mc)
            t = lo + i * LANES + lax.iota(jnp.int32, LANES)
            rows = jnp.where(t < total, vals, t & (P - 1)) + b * P
            idx_v[pl.ds(i * LANES, LANES)] = rows
            return jnp.max(vals)

        lax.fori_loop(0, rc * (CH // LANES), p2_body, maxc)

        bufs = (buf0, buf1)
        gsems = (gs0, gs1)
        wsems = (ws0, ws1)

        def idx_slice(c):
            return table_hbm.at[idx_v.at[pl.ds(c * CH, CH)]]

        def has_real(c):
            return c < rc

        def drain_real(c):
            @pl.when(has_real(c))
            def _():
                pltpu.make_async_copy(
                    zbuf, out_slice(s0 + c), wsems[c % 2]).wait()

        def start_chunk(c):
            @pl.when(has_real(c))
            def _():
                pltpu.async_copy(idx_slice(c), bufs[c % 2], gsems[c % 2])

        def finish_chunk(c):
            nreal = jnp.clip(total - (s0 + c) * CH, 0, CH)

            @pl.when(has_real(c))
            def _():
                pltpu.make_async_copy(
                    idx_slice(c), bufs[c % 2], gsems[c % 2]).wait()

                def ztail(r, _):
                    for j in range(C // LANES):
                        bufs[c % 2][r, pl.ds(j * LANES, LANES)] = jnp.zeros(
                            (LANES,), jnp.float32)
                    return 0

                lax.fori_loop(nreal, CH, ztail, 0)
                pltpu.async_copy(bufs[c % 2], out_slice(s0 + c), wsems[c % 2])

        for ch in range(NCHUNK):
            if ch >= 2:
                drain_real(ch - 2)
            start_chunk(ch)
            if ch >= 1:
                finish_chunk(ch - 1)
        finish_chunk(NCHUNK - 1)

        lax.fori_loop(n_pad_pre, n_pad, pad_issue, 0)
        drain_real(NCHUNK - 2)
        drain_real(NCHUNK - 1)

        def pad_wait(i, _):
            pltpu.make_async_copy(zbuf, out_slice(0), wsp).wait()
            return 0

        lax.fori_loop(0, n_pad, pad_wait, 0)

    return k(table, durations)


def kernel(encoder_output, log_durations):
    ld = log_durations[..., 0]
    m = (ld > 0).astype(jnp.int32)
    durations = (jnp.floor(jnp.power(2.0, ld) + 0.0001)
                 .astype(jnp.int32) * m)
    table = encoder_output.reshape(B * P, C)
    out = _sc_expand(table, durations.reshape(-1))
    return out.reshape(B, L, C)
